# baseline (speedup 1.0000x reference)
import functools
import math

import jax
import jax.numpy as jnp
from jax.experimental import pallas as pl
from jax.experimental.pallas import tpu as pltpu

B, L, Dm, Hq, Hkv, Dh, E, K, F = 1, 2048, 2048, 32, 4, 128, 16, 8, 128
EPS = 1e-06
NH = Hq + 2 * Hkv
REP = Hq // Hkv

BL = 256
BQ = 256
BT = 256


def _dot(a, b):
    return jax.lax.dot_general(a, b, (((1,), (0,)), ((), ())),
                               preferred_element_type=jnp.float32)


def _qkv_kernel(x_ref, w_ref, na_ref, normw_ref, cos_ref, sin_ref, out_ref):
    j = pl.program_id(1)
    x = x_ref[...]
    ms = jnp.mean(x * x, axis=-1, keepdims=True)
    h = x * jax.lax.rsqrt(ms + EPS) * na_ref[...]
    y = _dot(h, w_ref[...])

    @pl.when(j < Hq + Hkv)
    def _():
        msy = jnp.mean(y * y, axis=-1, keepdims=True)
        yn = y * jax.lax.rsqrt(msy + EPS) * normw_ref[...]
        y1 = yn[:, : Dh // 2]
        y2 = yn[:, Dh // 2:]
        rot = jnp.concatenate([-y2, y1], axis=-1)
        out_ref[0] = cos_ref[...] * yn + sin_ref[...] * rot

    @pl.when(j >= Hq + Hkv)
    def _():
        out_ref[0] = y


def _attn_kernel(q_ref, k_ref, v_ref, o_ref):
    i = pl.program_id(1)
    q = q_ref[0]
    k = k_ref[0]
    s = jax.lax.dot_general(q, k, (((1,), (1,)), ((), ())),
                            preferred_element_type=jnp.float32)
    s = s * (1.0 / math.sqrt(Dh))
    row = i * BQ + jax.lax.broadcasted_iota(jnp.int32, (BQ, L), 0)
    col = jax.lax.broadcasted_iota(jnp.int32, (BQ, L), 1)
    s = jnp.where(col <= row, s, jnp.float32(-1e30))
    m = jnp.max(s, axis=-1, keepdims=True)
    p = jnp.exp(s - m)
    l = jnp.sum(p, axis=-1, keepdims=True)
    o = _dot(p, v_ref[0]) / l
    o_ref[...] = o


def _oproj_gate_kernel(o_ref, wo_ref, x_ref, nm_ref, wg_ref,
                       xa_ref, h2_ref, wfull_ref):
    xa = x_ref[...] + _dot(o_ref[...], wo_ref[...])
    xa_ref[...] = xa
    ms = jnp.mean(xa * xa, axis=-1, keepdims=True)
    h2 = xa * jax.lax.rsqrt(ms + EPS) * nm_ref[...]
    h2_ref[...] = h2
    logits = _dot(h2, wg_ref[...])
    mx = jnp.max(logits, axis=-1, keepdims=True)
    ex = jnp.exp(logits - mx)
    probs = ex / jnp.sum(ex, axis=-1, keepdims=True)
    a = probs[:, None, :]
    b = probs[:, :, None]
    ij = jax.lax.broadcasted_iota(jnp.int32, (BT, E, E), 2)
    ie = jax.lax.broadcasted_iota(jnp.int32, (BT, E, E), 1)
    cmp = (a > b) | ((a == b) & (ij < ie))
    rank = jnp.sum(cmp.astype(jnp.float32), axis=2)
    w = jnp.where(rank < K, probs, 0.0)
    wfull_ref[...] = w / jnp.sum(w, axis=-1, keepdims=True)


def _moe_kernel(h2_ref, wg_ref, wu_ref, wd_ref, wfull_ref, xa_ref, out_ref):
    e = pl.program_id(1)
    h2 = h2_ref[...]
    g = _dot(h2, wg_ref[0])
    u = _dot(h2, wu_ref[0])
    act = g * jax.lax.logistic(g) * u
    lane = jax.lax.broadcasted_iota(jnp.int32, (BT, E), 1)
    wcol = jnp.sum(jnp.where(lane == e, wfull_ref[...], 0.0),
                   axis=1, keepdims=True)
    contrib = _dot(act * wcol, wd_ref[0])

    @pl.when(e == 0)
    def _():
        out_ref[...] = xa_ref[...] + contrib

    @pl.when(e > 0)
    def _():
        out_ref[...] += contrib


@functools.partial(jax.jit, static_argnames=("interpret",))
def kernel(x, cos, sin, Wq, Wk, Wv, Wo, qn, kn, na, nm, Wgate, Wg, Wu, Wd,
           interpret=False):
    f32 = jnp.float32
    x2 = x[0]
    cos2 = cos[0]
    sin2 = sin[0]
    wqkv = jnp.concatenate([Wq, Wk, Wv], axis=1)
    normw = jnp.concatenate(
        [jnp.tile(qn, Hq), jnp.tile(kn, Hkv), jnp.ones((Hkv * Dh,), f32)]
    ).reshape(1, NH * Dh)
    na2 = na.reshape(1, Dm)
    nm2 = nm.reshape(1, Dm)

    qkv = pl.pallas_call(
        _qkv_kernel,
        grid=(L // BL, NH),
        in_specs=[
            pl.BlockSpec((BL, Dm), lambda i, j: (i, 0)),
            pl.BlockSpec((Dm, Dh), lambda i, j: (0, j)),
            pl.BlockSpec((1, Dm), lambda i, j: (0, 0)),
            pl.BlockSpec((1, Dh), lambda i, j: (0, j)),
            pl.BlockSpec((BL, Dh), lambda i, j: (i, 0)),
            pl.BlockSpec((BL, Dh), lambda i, j: (i, 0)),
        ],
        out_specs=pl.BlockSpec((1, BL, Dh), lambda i, j: (j, i, 0)),
        out_shape=jax.ShapeDtypeStruct((NH, L, Dh), f32),
        interpret=interpret,
    )(x2, wqkv, na2, normw, cos2, sin2)

    q = qkv[:Hq]
    kk = qkv[Hq:Hq + Hkv]
    vv = qkv[Hq + Hkv:]

    o = pl.pallas_call(
        _attn_kernel,
        grid=(Hq, L // BQ),
        in_specs=[
            pl.BlockSpec((1, BQ, Dh), lambda h, i: (h, i, 0)),
            pl.BlockSpec((1, L, Dh), lambda h, i: (h // REP, 0, 0)),
            pl.BlockSpec((1, L, Dh), lambda h, i: (h // REP, 0, 0)),
        ],
        out_specs=pl.BlockSpec((BQ, Dh), lambda h, i: (i, h)),
        out_shape=jax.ShapeDtypeStruct((L, Hq * Dh), f32),
        interpret=interpret,
    )(q, kk, vv)

    xa, h2, wfull = pl.pallas_call(
        _oproj_gate_kernel,
        grid=(L // BT,),
        in_specs=[
            pl.BlockSpec((BT, Hq * Dh), lambda i: (i, 0)),
            pl.BlockSpec((Hq * Dh, Dm), lambda i: (0, 0)),
            pl.BlockSpec((BT, Dm), lambda i: (i, 0)),
            pl.BlockSpec((1, Dm), lambda i: (0, 0)),
            pl.BlockSpec((Dm, E), lambda i: (0, 0)),
        ],
        out_specs=[
            pl.BlockSpec((BT, Dm), lambda i: (i, 0)),
            pl.BlockSpec((BT, Dm), lambda i: (i, 0)),
            pl.BlockSpec((BT, E), lambda i: (i, 0)),
        ],
        out_shape=[
            jax.ShapeDtypeStruct((L, Dm), f32),
            jax.ShapeDtypeStruct((L, Dm), f32),
            jax.ShapeDtypeStruct((L, E), f32),
        ],
        interpret=interpret,
    )(o, Wo, x2, nm2, Wgate)

    out = pl.pallas_call(
        _moe_kernel,
        grid=(L // BT, E),
        in_specs=[
            pl.BlockSpec((BT, Dm), lambda i, e: (i, 0)),
            pl.BlockSpec((1, Dm, F), lambda i, e: (e, 0, 0)),
            pl.BlockSpec((1, Dm, F), lambda i, e: (e, 0, 0)),
            pl.BlockSpec((1, F, Dm), lambda i, e: (e, 0, 0)),
            pl.BlockSpec((BT, E), lambda i, e: (i, 0)),
            pl.BlockSpec((BT, Dm), lambda i, e: (i, 0)),
        ],
        out_specs=pl.BlockSpec((BT, Dm), lambda i, e: (i, 0)),
        out_shape=jax.ShapeDtypeStruct((L, Dm), f32),
        interpret=interpret,
    )(h2, Wg, Wu, Wd, wfull, xa)

    return out.reshape(B, L, Dm)


# speedup vs baseline: 1.1536x; 1.1536x over previous
import functools
import math

import jax
import jax.numpy as jnp
from jax.experimental import pallas as pl
from jax.experimental.pallas import tpu as pltpu

B, L, Dm, Hq, Hkv, Dh, E, K, F = 1, 2048, 2048, 32, 4, 128, 16, 8, 128
EPS = 1e-06
NH = Hq + 2 * Hkv
REP = Hq // Hkv

BL = 512
BQ = 256
BK = 256
BT = 256


def _dot(a, b):
    return jax.lax.dot_general(a, b, (((1,), (0,)), ((), ())),
                               preferred_element_type=jnp.float32)


def _dot_nt(a, b):
    return jax.lax.dot_general(a, b, (((1,), (1,)), ((), ())),
                               preferred_element_type=jnp.float32)


def _qkv_kernel(x_ref, w_ref, na_ref, normw_ref, cos_ref, sin_ref, out_ref,
                h_scr):
    j = pl.program_id(1)

    @pl.when(j == 0)
    def _():
        x = x_ref[...]
        ms = jnp.mean(x * x, axis=-1, keepdims=True)
        h_scr[...] = (x * jax.lax.rsqrt(ms + EPS) * na_ref[...]
                      ).astype(jnp.bfloat16)

    y = _dot(h_scr[...], w_ref[...])

    @pl.when(j < Hq + Hkv)
    def _():
        msy = jnp.mean(y * y, axis=-1, keepdims=True)
        yn = y * jax.lax.rsqrt(msy + EPS) * normw_ref[...]
        y1 = yn[:, : Dh // 2]
        y2 = yn[:, Dh // 2:]
        rot = jnp.concatenate([-y2, y1], axis=-1)
        r = cos_ref[...] * yn + sin_ref[...] * rot
        r = jnp.where(j < Hq, r * (1.0 / math.sqrt(Dh)), r)
        out_ref[0] = r.astype(jnp.bfloat16)

    @pl.when(j >= Hq + Hkv)
    def _():
        out_ref[0] = y.astype(jnp.bfloat16)


def _attn_kernel(q_ref, k_ref, v_ref, o_ref):
    i = pl.program_id(1)
    q = q_ref[0]
    rowid = i * BQ + jax.lax.broadcasted_iota(jnp.int32, (BQ, BK), 0)
    colid = jax.lax.broadcasted_iota(jnp.int32, (BQ, BK), 1)

    def body(j, carry):
        acc, m, l = carry
        kb = k_ref[0, pl.ds(j * BK, BK), :]
        vb = v_ref[0, pl.ds(j * BK, BK), :]
        s = _dot_nt(q, kb)
        s = jnp.where(j * BK + colid <= rowid, s, jnp.float32(-1e30))
        m_new = jnp.maximum(m, jnp.max(s, axis=-1, keepdims=True))
        p = jnp.exp(s - m_new)
        corr = jnp.exp(m - m_new)
        l_new = l * corr + jnp.sum(p, axis=-1, keepdims=True)
        acc_new = acc * corr + _dot(p.astype(jnp.bfloat16), vb)
        return acc_new, m_new, l_new

    acc, m, l = jax.lax.fori_loop(
        0, i + 1, body,
        (jnp.zeros((BQ, Dh), jnp.float32),
         jnp.full((BQ, 1), -jnp.inf, jnp.float32),
         jnp.zeros((BQ, 1), jnp.float32)))
    o_ref[...] = (acc / l).astype(jnp.bfloat16)


def _oproj_gate_kernel(o_ref, wo_ref, x_ref, nm_ref, wg_ref,
                       xa_ref, h2_ref, wfull_ref):
    xa = x_ref[...] + _dot(o_ref[...], wo_ref[...])
    xa_ref[...] = xa
    ms = jnp.mean(xa * xa, axis=-1, keepdims=True)
    h2 = xa * jax.lax.rsqrt(ms + EPS) * nm_ref[...]
    h2_ref[...] = h2.astype(jnp.bfloat16)
    logits = _dot(h2, wg_ref[...])
    mx = jnp.max(logits, axis=-1, keepdims=True)
    ex = jnp.exp(logits - mx)
    probs = ex / jnp.sum(ex, axis=-1, keepdims=True)
    a = probs[:, None, :]
    b = probs[:, :, None]
    ij = jax.lax.broadcasted_iota(jnp.int32, (BT, E, E), 2)
    ie = jax.lax.broadcasted_iota(jnp.int32, (BT, E, E), 1)
    cmp = (a > b) | ((a == b) & (ij < ie))
    rank = jnp.sum(cmp.astype(jnp.float32), axis=2)
    w = jnp.where(rank < K, probs, 0.0)
    wfull_ref[...] = w / jnp.sum(w, axis=-1, keepdims=True)


def _moe_kernel(h2_ref, wg_ref, wu_ref, wd_ref, wfull_ref, sel_ref, xa_ref,
                out_ref):
    h2 = h2_ref[...]
    g = _dot(h2, wg_ref[...])
    u = _dot(h2, wu_ref[...])
    act = g * jax.lax.logistic(g) * u
    wexp = _dot(wfull_ref[...], sel_ref[...])
    down_in = (act * wexp).astype(jnp.bfloat16)
    out_ref[...] = xa_ref[...] + _dot(down_in, wd_ref[...])


@functools.partial(jax.jit, static_argnames=("interpret",))
def kernel(x, cos, sin, Wq, Wk, Wv, Wo, qn, kn, na, nm, Wgate, Wg, Wu, Wd,
           interpret=False):
    f32 = jnp.float32
    bf16 = jnp.bfloat16
    x2 = x[0]
    cos2 = cos[0]
    sin2 = sin[0]
    wqkv = jnp.concatenate([Wq, Wk, Wv], axis=1).astype(bf16)
    normw = jnp.concatenate(
        [jnp.tile(qn, Hq), jnp.tile(kn, Hkv), jnp.ones((Hkv * Dh,), f32)]
    ).reshape(1, NH * Dh)
    na2 = na.reshape(1, Dm)
    nm2 = nm.reshape(1, Dm)
    wo_b = Wo.astype(bf16)
    wg_all = jnp.transpose(Wg, (1, 0, 2)).reshape(Dm, E * F).astype(bf16)
    wu_all = jnp.transpose(Wu, (1, 0, 2)).reshape(Dm, E * F).astype(bf16)
    wd_all = Wd.reshape(E * F, Dm).astype(bf16)
    sel = jnp.repeat(jnp.eye(E, dtype=f32), F, axis=1).reshape(E, E * F)

    qkv = pl.pallas_call(
        _qkv_kernel,
        grid=(L // BL, NH),
        in_specs=[
            pl.BlockSpec((BL, Dm), lambda i, j: (i, 0)),
            pl.BlockSpec((Dm, Dh), lambda i, j: (0, j)),
            pl.BlockSpec((1, Dm), lambda i, j: (0, 0)),
            pl.BlockSpec((1, Dh), lambda i, j: (0, j)),
            pl.BlockSpec((BL, Dh), lambda i, j: (i, 0)),
            pl.BlockSpec((BL, Dh), lambda i, j: (i, 0)),
        ],
        out_specs=pl.BlockSpec((1, BL, Dh), lambda i, j: (j, i, 0)),
        out_shape=jax.ShapeDtypeStruct((NH, L, Dh), bf16),
        scratch_shapes=[pltpu.VMEM((BL, Dm), bf16)],
        interpret=interpret,
    )(x2, wqkv, na2, normw, cos2, sin2)

    o = pl.pallas_call(
        _attn_kernel,
        grid=(Hq, L // BQ),
        in_specs=[
            pl.BlockSpec((1, BQ, Dh), lambda h, i: (h, i, 0)),
            pl.BlockSpec((1, L, Dh), lambda h, i: (Hq + h // REP, 0, 0)),
            pl.BlockSpec((1, L, Dh), lambda h, i: (Hq + Hkv + h // REP, 0, 0)),
        ],
        out_specs=pl.BlockSpec((BQ, Dh), lambda h, i: (i, h)),
        out_shape=jax.ShapeDtypeStruct((L, Hq * Dh), bf16),
        interpret=interpret,
    )(qkv, qkv, qkv)

    xa, h2, wfull = pl.pallas_call(
        _oproj_gate_kernel,
        grid=(L // BT,),
        in_specs=[
            pl.BlockSpec((BT, Hq * Dh), lambda i: (i, 0)),
            pl.BlockSpec((Hq * Dh, Dm), lambda i: (0, 0)),
            pl.BlockSpec((BT, Dm), lambda i: (i, 0)),
            pl.BlockSpec((1, Dm), lambda i: (0, 0)),
            pl.BlockSpec((Dm, E), lambda i: (0, 0)),
        ],
        out_specs=[
            pl.BlockSpec((BT, Dm), lambda i: (i, 0)),
            pl.BlockSpec((BT, Dm), lambda i: (i, 0)),
            pl.BlockSpec((BT, E), lambda i: (i, 0)),
        ],
        out_shape=[
            jax.ShapeDtypeStruct((L, Dm), f32),
            jax.ShapeDtypeStruct((L, Dm), bf16),
            jax.ShapeDtypeStruct((L, E), f32),
        ],
        interpret=interpret,
    )(o, wo_b, x2, nm2, Wgate)

    out = pl.pallas_call(
        _moe_kernel,
        grid=(L // BT,),
        in_specs=[
            pl.BlockSpec((BT, Dm), lambda i: (i, 0)),
            pl.BlockSpec((Dm, E * F), lambda i: (0, 0)),
            pl.BlockSpec((Dm, E * F), lambda i: (0, 0)),
            pl.BlockSpec((E * F, Dm), lambda i: (0, 0)),
            pl.BlockSpec((BT, E), lambda i: (i, 0)),
            pl.BlockSpec((E, E * F), lambda i: (0, 0)),
            pl.BlockSpec((BT, Dm), lambda i: (i, 0)),
        ],
        out_specs=pl.BlockSpec((BT, Dm), lambda i: (i, 0)),
        out_shape=jax.ShapeDtypeStruct((L, Dm), f32),
        interpret=interpret,
    )(h2, wg_all, wu_all, wd_all, wfull, sel, xa)

    return out.reshape(B, L, Dm)


# speedup vs baseline: 1.7660x; 1.5308x over previous
import functools
import math

import jax
import jax.numpy as jnp
from jax.experimental import pallas as pl
from jax.experimental.pallas import tpu as pltpu

B, L, Dm, Hq, Hkv, Dh, E, K, F = 1, 2048, 2048, 32, 4, 128, 16, 8, 128
EPS = 1e-06
NH = Hq + 2 * Hkv
REP = Hq // Hkv

BL = 512
BQ = 512
BK = 512
BT = 256
VA = 2 * Dh


def _dot(a, b):
    return jax.lax.dot_general(a, b, (((1,), (0,)), ((), ())),
                               preferred_element_type=jnp.float32)


def _dot_nt(a, b):
    return jax.lax.dot_general(a, b, (((1,), (1,)), ((), ())),
                               preferred_element_type=jnp.float32)


def _qkv_kernel(x_ref, w_ref, na_ref, normw_ref, cos_ref, sins_ref, out_ref,
                h_scr):
    j = pl.program_id(1)

    @pl.when(j == 0)
    def _():
        x = x_ref[...]
        ms = jnp.mean(x * x, axis=-1, keepdims=True)
        h_scr[...] = (x * jax.lax.rsqrt(ms + EPS) * na_ref[...]
                      ).astype(jnp.bfloat16)

    y = _dot(h_scr[...], w_ref[...])

    @pl.when(j < Hq + Hkv)
    def _():
        jm = jnp.full((Dh, Dh), 1.0 / Dh, jnp.bfloat16)
        msy = _dot((y * y).astype(jnp.bfloat16), jm)
        yn = y * jax.lax.rsqrt(msy + EPS) * normw_ref[...]
        rot = jnp.roll(yn, -(Dh // 2), axis=1)
        out_ref[0] = (cos_ref[...] * yn + sins_ref[...] * rot
                      ).astype(jnp.bfloat16)

    @pl.when(j >= Hq + Hkv)
    def _():
        out_ref[0] = y.astype(jnp.bfloat16)


def _attn_kernel(q_ref, k_ref, vaug_ref, o_ref):
    i = pl.program_id(1)
    q = q_ref[0]

    def body(j, acc):
        kb = k_ref[0, pl.ds(j * BK, BK), :]
        p = jnp.exp(_dot_nt(q, kb)).astype(jnp.bfloat16)
        vb = vaug_ref[0, pl.ds(j * BK, BK), :]
        return acc + _dot(p, vb)

    acc = jax.lax.fori_loop(0, i, body, jnp.zeros((BQ, VA), jnp.float32))
    kb = k_ref[0, pl.ds(i * BK, BK), :]
    s = _dot_nt(q, kb)
    rowid = jax.lax.broadcasted_iota(jnp.int32, (BQ, BK), 0)
    colid = jax.lax.broadcasted_iota(jnp.int32, (BQ, BK), 1)
    p = jnp.where(colid <= rowid, jnp.exp(s), 0.0).astype(jnp.bfloat16)
    acc = acc + _dot(p, vaug_ref[0, pl.ds(i * BK, BK), :])
    o_ref[...] = (acc[:, :Dh] / acc[:, Dh:Dh + 1]).astype(jnp.bfloat16)


def _oproj_gate_kernel(o_ref, wo_ref, x_ref, nm_ref, wg_ref,
                       xa_ref, h2_ref, wfull_ref):
    xa = x_ref[...] + _dot(o_ref[...], wo_ref[...])
    xa_ref[...] = xa
    ms = jnp.mean(xa * xa, axis=-1, keepdims=True)
    h2 = xa * jax.lax.rsqrt(ms + EPS) * nm_ref[...]
    h2_ref[...] = h2.astype(jnp.bfloat16)
    logits = _dot(h2, wg_ref[...])
    mx = jnp.max(logits, axis=-1, keepdims=True)
    ex = jnp.exp(logits - mx)
    probs = ex / jnp.sum(ex, axis=-1, keepdims=True)
    a = probs[:, None, :]
    b = probs[:, :, None]
    ij = jax.lax.broadcasted_iota(jnp.int32, (BT, E, E), 2)
    ie = jax.lax.broadcasted_iota(jnp.int32, (BT, E, E), 1)
    cmp = (a > b) | ((a == b) & (ij < ie))
    rank = jnp.sum(cmp.astype(jnp.float32), axis=2)
    w = jnp.where(rank < K, probs, 0.0)
    wfull_ref[...] = w / jnp.sum(w, axis=-1, keepdims=True)


def _moe_kernel(h2_ref, wg_ref, wu_ref, wd_ref, wfull_ref, sel_ref, xa_ref,
                out_ref):
    h2 = h2_ref[...]
    g = _dot(h2, wg_ref[...])
    u = _dot(h2, wu_ref[...])
    act = g * jax.lax.logistic(g) * u
    wexp = _dot(wfull_ref[...], sel_ref[...])
    down_in = (act * wexp).astype(jnp.bfloat16)
    out_ref[...] = xa_ref[...] + _dot(down_in, wd_ref[...])


@functools.partial(jax.jit, static_argnames=("interpret",))
def kernel(x, cos, sin, Wq, Wk, Wv, Wo, qn, kn, na, nm, Wgate, Wg, Wu, Wd,
           interpret=False):
    f32 = jnp.float32
    bf16 = jnp.bfloat16
    x2 = x[0]
    cos2 = cos[0]
    sign = jnp.concatenate([-jnp.ones((Dh // 2,), f32),
                            jnp.ones((Dh // 2,), f32)])
    sins2 = sin[0] * sign
    wqkv = jnp.concatenate([Wq, Wk, Wv], axis=1).astype(bf16)
    normw = jnp.concatenate(
        [jnp.tile(qn * (1.0 / math.sqrt(Dh)), Hq), jnp.tile(kn, Hkv),
         jnp.ones((Hkv * Dh,), f32)]).reshape(1, NH * Dh)
    na2 = na.reshape(1, Dm)
    nm2 = nm.reshape(1, Dm)
    wo_b = Wo.astype(bf16)
    wg_all = jnp.transpose(Wg, (1, 0, 2)).reshape(Dm, E * F).astype(bf16)
    wu_all = jnp.transpose(Wu, (1, 0, 2)).reshape(Dm, E * F).astype(bf16)
    wd_all = Wd.reshape(E * F, Dm).astype(bf16)
    sel = jnp.repeat(jnp.eye(E, dtype=f32), F, axis=1).reshape(E, E * F)

    qkv = pl.pallas_call(
        _qkv_kernel,
        grid=(L // BL, NH),
        in_specs=[
            pl.BlockSpec((BL, Dm), lambda i, j: (i, 0)),
            pl.BlockSpec((Dm, Dh), lambda i, j: (0, j)),
            pl.BlockSpec((1, Dm), lambda i, j: (0, 0)),
            pl.BlockSpec((1, Dh), lambda i, j: (0, j)),
            pl.BlockSpec((BL, Dh), lambda i, j: (i, 0)),
            pl.BlockSpec((BL, Dh), lambda i, j: (i, 0)),
        ],
        out_specs=pl.BlockSpec((1, BL, Dh), lambda i, j: (j, i, 0)),
        out_shape=jax.ShapeDtypeStruct((NH, L, Dh), bf16),
        scratch_shapes=[pltpu.VMEM((BL, Dm), bf16)],
        compiler_params=pltpu.CompilerParams(
            dimension_semantics=("parallel", "arbitrary")),
        interpret=interpret,
    )(x2, wqkv, na2, normw, cos2, sins2)

    vv = qkv[Hq + Hkv:]
    vaug = jnp.concatenate(
        [vv, jnp.ones((Hkv, L, 1), bf16), jnp.zeros((Hkv, L, VA - Dh - 1), bf16)],
        axis=2)

    o = pl.pallas_call(
        _attn_kernel,
        grid=(Hq, L // BQ),
        in_specs=[
            pl.BlockSpec((1, BQ, Dh), lambda h, i: (h, i, 0)),
            pl.BlockSpec((1, L, Dh), lambda h, i: (Hq + h // REP, 0, 0)),
            pl.BlockSpec((1, L, VA), lambda h, i: (h // REP, 0, 0)),
        ],
        out_specs=pl.BlockSpec((BQ, Dh), lambda h, i: (i, h)),
        out_shape=jax.ShapeDtypeStruct((L, Hq * Dh), bf16),
        compiler_params=pltpu.CompilerParams(
            dimension_semantics=("parallel", "parallel")),
        interpret=interpret,
    )(qkv, qkv, vaug)

    xa, h2, wfull = pl.pallas_call(
        _oproj_gate_kernel,
        grid=(L // BT,),
        in_specs=[
            pl.BlockSpec((BT, Hq * Dh), lambda i: (i, 0)),
            pl.BlockSpec((Hq * Dh, Dm), lambda i: (0, 0)),
            pl.BlockSpec((BT, Dm), lambda i: (i, 0)),
            pl.BlockSpec((1, Dm), lambda i: (0, 0)),
            pl.BlockSpec((Dm, E), lambda i: (0, 0)),
        ],
        out_specs=[
            pl.BlockSpec((BT, Dm), lambda i: (i, 0)),
            pl.BlockSpec((BT, Dm), lambda i: (i, 0)),
            pl.BlockSpec((BT, E), lambda i: (i, 0)),
        ],
        out_shape=[
            jax.ShapeDtypeStruct((L, Dm), f32),
            jax.ShapeDtypeStruct((L, Dm), bf16),
            jax.ShapeDtypeStruct((L, E), f32),
        ],
        compiler_params=pltpu.CompilerParams(
            dimension_semantics=("parallel",)),
        interpret=interpret,
    )(o, wo_b, x2, nm2, Wgate)

    out = pl.pallas_call(
        _moe_kernel,
        grid=(L // BT,),
        in_specs=[
            pl.BlockSpec((BT, Dm), lambda i: (i, 0)),
            pl.BlockSpec((Dm, E * F), lambda i: (0, 0)),
            pl.BlockSpec((Dm, E * F), lambda i: (0, 0)),
            pl.BlockSpec((E * F, Dm), lambda i: (0, 0)),
            pl.BlockSpec((BT, E), lambda i: (i, 0)),
            pl.BlockSpec((E, E * F), lambda i: (0, 0)),
            pl.BlockSpec((BT, Dm), lambda i: (i, 0)),
        ],
        out_specs=pl.BlockSpec((BT, Dm), lambda i: (i, 0)),
        out_shape=jax.ShapeDtypeStruct((L, Dm), f32),
        compiler_params=pltpu.CompilerParams(
            dimension_semantics=("parallel",)),
        interpret=interpret,
    )(h2, wg_all, wu_all, wd_all, wfull, sel, xa)

    return out.reshape(B, L, Dm)
